# baseline (speedup 1.0000x reference)
import jax
import jax.numpy as jnp
from jax.experimental import pallas as pl

NUM_CLASSES = 1000
ROWS = 1024 * 50
W = 1024
FLAT_ROWS = ROWS * NUM_CLASSES // W
BLOCK_ROWS = 1000
GRID = FLAT_ROWS // BLOCK_ROWS


def _onehot_block(t0_ref, t1_ref, t2_ref, o_ref):
    j = jax.lax.broadcasted_iota(jnp.int32, (BLOCK_ROWS, W), 1)
    hit = (j == t0_ref[0]) | (j == t1_ref[0]) | (j == t2_ref[0])
    o_ref[...] = hit.astype(jnp.int32)


def kernel(x):
    xf = x.reshape(ROWS)
    i = jnp.arange(FLAT_ROWS, dtype=jnp.int32)
    p0 = i * W
    r0 = p0 // NUM_CLASSES
    c0 = p0 - r0 * NUM_CLASSES

    def target(k):
        r = r0 + k
        xr = xf[jnp.minimum(r, ROWS - 1)]
        t = xr + k * NUM_CLASSES - c0
        valid = (r < ROWS) & (t >= 0) & (t < W)
        return jnp.where(valid, t, -1).reshape(GRID, BLOCK_ROWS, 1)

    t0, t1, t2 = target(0), target(1), target(2)

    out = pl.pallas_call(
        _onehot_block,
        grid=(GRID,),
        in_specs=[pl.BlockSpec((1, BLOCK_ROWS, 1), lambda g: (g, 0, 0))] * 3,
        out_specs=pl.BlockSpec((BLOCK_ROWS, W), lambda g: (g, 0)),
        out_shape=jax.ShapeDtypeStruct((FLAT_ROWS, W), jnp.int32),
    )(t0, t1, t2)
    return out.reshape(1024, 50, NUM_CLASSES)


# speedup vs baseline: 3.1481x; 3.1481x over previous
import jax
import jax.numpy as jnp
from jax.experimental import pallas as pl

NUM_CLASSES = 1000
B = 1024
S = 50
BLOCK_B = 16


def _onehot_block(x_ref, o_ref):
    j = jax.lax.broadcasted_iota(jnp.int32, (BLOCK_B, S, NUM_CLASSES), 2)
    o_ref[...] = (j == x_ref[...]).astype(jnp.int32)


def kernel(x):
    x3 = x.reshape(B, S, 1)
    return pl.pallas_call(
        _onehot_block,
        grid=(B // BLOCK_B,),
        in_specs=[pl.BlockSpec((BLOCK_B, S, 1), lambda g: (g, 0, 0))],
        out_specs=pl.BlockSpec((BLOCK_B, S, NUM_CLASSES), lambda g: (g, 0, 0)),
        out_shape=jax.ShapeDtypeStruct((B, S, NUM_CLASSES), jnp.int32),
    )(x3)


# speedup vs baseline: 16.4079x; 5.2121x over previous
import jax
import jax.numpy as jnp
from jax.experimental import pallas as pl

NUM_CLASSES = 1000
B = 1024
S = 50


def _onehot_block(x_ref, o_ref):
    c = jax.lax.broadcasted_iota(jnp.int32, (1, NUM_CLASSES, B), 1)
    o_ref[...] = (c == x_ref[...]).astype(jnp.int32)


def kernel(x):
    xt = x.T.reshape(S, 1, B)
    out_t = pl.pallas_call(
        _onehot_block,
        grid=(S,),
        in_specs=[pl.BlockSpec((1, 1, B), lambda s: (s, 0, 0))],
        out_specs=pl.BlockSpec((1, NUM_CLASSES, B), lambda s: (s, 0, 0)),
        out_shape=jax.ShapeDtypeStruct((S, NUM_CLASSES, B), jnp.int32),
    )(xt)
    return jnp.transpose(out_t, (2, 0, 1))
